# baseline (speedup 1.0000x reference)
import functools

import jax
import jax.numpy as jnp
from jax import lax
from jax.experimental import pallas as pl
from jax.experimental.pallas import tpu as pltpu
from jax.experimental.pallas import tpu_sc as plsc

N = 10000
E = 320000
D = 128
H = 128
C = 10
G = 64

NC = 2
NS = 16
NW = NC * NS
CHUNK = 64
NCHUNK = 160
NB = 5
EPW = CHUNK * NCHUNK
EPAD = EPW * NW
NPAD = 10240
RPS = NPAD // NS
ZROWS = 32

BN = 1000
GRID = N // BN


def _sc_segsum(h, src3, dst3):
    mesh = plsc.VectorSubcoreMesh(core_axis_name="c", subcore_axis_name="s")

    @functools.partial(
        pl.kernel,
        out_type=jax.ShapeDtypeStruct((NC, N, D), jnp.float32),
        mesh=mesh,
        scratch_types=[
            pltpu.VMEM((NB, CHUNK), jnp.int32),
            pltpu.VMEM((NB, CHUNK), jnp.int32),
            pltpu.VMEM((NB, CHUNK, D), jnp.float32),
            pltpu.VMEM((ZROWS, D), jnp.float32),
            pltpu.VMEM_SHARED((NPAD, D), jnp.float32),
        ]
        + [pltpu.SemaphoreType.DMA] * (4 * NB),
    )
    def k(h_hbm, src_hbm, dst_hbm, out_hbm, srcv, dstv, rows_v, zbuf, acc_sh,
          *sems):
        gsem = sems[:NB]
        ssem = sems[NB:2 * NB]
        isem_s = sems[2 * NB:3 * NB]
        isem_d = sems[3 * NB:]
        cid = lax.axis_index("c")
        sid = lax.axis_index("s")
        wid = sid * NC + cid

        zbuf[...] = jnp.zeros((ZROWS, D), jnp.float32)

        @pl.loop(0, RPS, step=ZROWS)
        def _(r):
            pltpu.sync_copy(zbuf, acc_sh.at[pl.ds(sid * RPS + r, ZROWS)])

        plsc.subcore_barrier()

        def is_start(b, ci):
            pltpu.async_copy(src_hbm.at[wid].at[ci], srcv.at[b], isem_s[b])

        def is_wait(b, ci):
            pltpu.make_async_copy(src_hbm.at[wid].at[ci], srcv.at[b],
                                  isem_s[b]).wait()

        def id_start(b, ci):
            pltpu.async_copy(dst_hbm.at[wid].at[ci], dstv.at[b], isem_d[b])

        def id_wait(b, ci):
            pltpu.make_async_copy(dst_hbm.at[wid].at[ci], dstv.at[b],
                                  isem_d[b]).wait()

        def g_start(b):
            pltpu.async_copy(h_hbm.at[srcv.at[b]], rows_v.at[b], gsem[b])

        def g_wait(b):
            pltpu.make_async_copy(h_hbm.at[srcv.at[b]], rows_v.at[b],
                                  gsem[b]).wait()

        def s_start(b):
            pltpu.async_copy(rows_v.at[b], acc_sh.at[dstv.at[b]], ssem[b],
                             add=True)

        def s_wait(b):
            pltpu.make_async_copy(rows_v.at[b], acc_sh.at[dstv.at[b]],
                                  ssem[b]).wait()

        for b in range(NB):
            is_start(b, b)
            id_start(b, b)
        for b in range(NB):
            is_wait(b, b)
            g_start(b)

        @pl.loop(0, NCHUNK - NB, step=NB)
        def _(c0):
            for b in range(NB):
                g_wait(b)
                is_start(b, c0 + NB + b)
                id_wait(b, c0 + b)
                s_start(b)
            for b in range(NB):
                s_wait(b)
                id_start(b, c0 + NB + b)
                is_wait(b, c0 + NB + b)
                g_start(b)

        for b in range(NB):
            g_wait(b)
            id_wait(b, NCHUNK - NB + b)
            s_start(b)
        for b in range(NB):
            s_wait(b)

        plsc.subcore_barrier()

        @pl.when(sid < NS - 1)
        def _():
            pltpu.sync_copy(acc_sh.at[pl.ds(sid * RPS, RPS)],
                            out_hbm.at[cid].at[pl.ds(sid * RPS, RPS)])

        @pl.when(sid == NS - 1)
        def _():
            pltpu.sync_copy(acc_sh.at[pl.ds((NS - 1) * RPS, N - (NS - 1) * RPS)],
                            out_hbm.at[cid].at[pl.ds((NS - 1) * RPS, N - (NS - 1) * RPS)])

    return k(h, src3, dst3)


def _dot(a, b_t):
    return lax.dot_general(a, b_t, (((1,), (1,)), ((), ())),
                           precision=lax.Precision.HIGHEST,
                           preferred_element_type=jnp.float32)


def _tc_layer(agg, h, w_rel, b_rel, w_root, relu):
    def body(agg_ref, h_ref, wr_ref, b_ref, wo_ref, o_ref):
        a = agg_ref[0] + agg_ref[1]
        o = _dot(a, wr_ref[...]) + _dot(h_ref[...], wo_ref[...]) + b_ref[...]
        if relu:
            o = jnp.maximum(o, 0.0)
        o_ref[...] = o

    return pl.pallas_call(
        body,
        grid=(GRID,),
        in_specs=[
            pl.BlockSpec((2, BN, D), lambda i: (0, i, 0)),
            pl.BlockSpec((BN, D), lambda i: (i, 0)),
            pl.BlockSpec((H, D), lambda i: (0, 0)),
            pl.BlockSpec((1, H), lambda i: (0, 0)),
            pl.BlockSpec((H, D), lambda i: (0, 0)),
        ],
        out_specs=pl.BlockSpec((BN, H), lambda i: (i, 0)),
        out_shape=jax.ShapeDtypeStruct((N, H), jnp.float32),
    )(agg, h, w_rel, b_rel.reshape(1, H), w_root)


def _tc_final(agg, h, w_rel, b_rel, w_root, batch, lin_w, lin_b):
    def body(agg_ref, h_ref, wr_ref, b_ref, wo_ref, batch_ref, lw_ref, lb_ref,
             o_ref, sums, cnts):
        i = pl.program_id(0)
        a = agg_ref[0] + agg_ref[1]
        h3 = _dot(a, wr_ref[...]) + _dot(h_ref[...], wo_ref[...]) + b_ref[...]
        seg = batch_ref[...]
        onehot = (seg == lax.broadcasted_iota(jnp.int32, (BN, G), 1)
                  ).astype(jnp.float32)
        part = lax.dot_general(onehot, h3, (((0,), (0,)), ((), ())),
                               precision=lax.Precision.HIGHEST,
                               preferred_element_type=jnp.float32)
        pcnt = lax.dot_general(onehot, jnp.ones((BN, D), jnp.float32),
                               (((0,), (0,)), ((), ())),
                               precision=lax.Precision.HIGHEST,
                               preferred_element_type=jnp.float32)

        @pl.when(i == 0)
        def _():
            sums[...] = part
            cnts[...] = pcnt

        @pl.when(i > 0)
        def _():
            sums[...] += part
            cnts[...] += pcnt

        @pl.when(i == GRID - 1)
        def _():
            pooled = sums[...] / jnp.maximum(cnts[...], 1.0)
            o_ref[...] = _dot(pooled, lw_ref[...]) + lb_ref[...]

    return pl.pallas_call(
        body,
        grid=(GRID,),
        in_specs=[
            pl.BlockSpec((2, BN, D), lambda i: (0, i, 0)),
            pl.BlockSpec((BN, D), lambda i: (i, 0)),
            pl.BlockSpec((H, D), lambda i: (0, 0)),
            pl.BlockSpec((1, H), lambda i: (0, 0)),
            pl.BlockSpec((H, D), lambda i: (0, 0)),
            pl.BlockSpec((BN, 1), lambda i: (i, 0)),
            pl.BlockSpec((C, H), lambda i: (0, 0)),
            pl.BlockSpec((1, C), lambda i: (0, 0)),
        ],
        out_specs=pl.BlockSpec((G, C), lambda i: (0, 0)),
        out_shape=jax.ShapeDtypeStruct((G, C), jnp.float32),
        scratch_shapes=[
            pltpu.VMEM((G, D), jnp.float32),
            pltpu.VMEM((G, D), jnp.float32),
        ],
    )(agg, h, w_rel, b_rel.reshape(1, H), w_root, batch.reshape(N, 1),
      lin_w, lin_b.reshape(1, C))


def kernel(x, edge_index, batch, W1_rel, b1_rel, W1_root, W2_rel, b2_rel,
           W2_root, W3_rel, b3_rel, W3_root, lin_W, lin_b):
    src = edge_index[0]
    dst = edge_index[1]
    pad = EPAD - E
    trash = N + jax.lax.iota(jnp.int32, pad) % (NPAD - N)
    src3 = jnp.concatenate([src, jnp.zeros((pad,), jnp.int32)]
                           ).reshape(NW, NCHUNK, CHUNK)
    dst3 = jnp.concatenate([dst, trash]).reshape(NW, NCHUNK, CHUNK)

    agg1 = _sc_segsum(x, src3, dst3)
    h1 = _tc_layer(agg1, x, W1_rel, b1_rel, W1_root, relu=True)
    agg2 = _sc_segsum(h1, src3, dst3)
    h2 = _tc_layer(agg2, h1, W2_rel, b2_rel, W2_root, relu=True)
    agg3 = _sc_segsum(h2, src3, dst3)
    return _tc_final(agg3, h2, W3_rel, b3_rel, W3_root, batch, lin_W, lin_b)


# speedup vs baseline: 1.0261x; 1.0261x over previous
import functools

import jax
import jax.numpy as jnp
from jax import lax
from jax.experimental import pallas as pl
from jax.experimental.pallas import tpu as pltpu
from jax.experimental.pallas import tpu_sc as plsc

N = 10000
E = 320000
D = 128
H = 128
C = 10
G = 64

NC = 2
NS = 16
NW = NC * NS
CHUNK = 80
NCHUNK = 128
NB = 4
EPW = CHUNK * NCHUNK
EPAD = EPW * NW
NPAD = 10240
RPS = NPAD // NS
ZROWS = 32

BN = 1000
GRID = N // BN


def _sc_segsum(h, src3, dst3):
    mesh = plsc.VectorSubcoreMesh(core_axis_name="c", subcore_axis_name="s")

    @functools.partial(
        pl.kernel,
        out_type=jax.ShapeDtypeStruct((NC, N, D), jnp.float32),
        mesh=mesh,
        scratch_types=[
            pltpu.VMEM((NB, CHUNK), jnp.int32),
            pltpu.VMEM((NB, CHUNK), jnp.int32),
            pltpu.VMEM((NB, CHUNK, D), jnp.float32),
            pltpu.VMEM((ZROWS, D), jnp.float32),
            pltpu.VMEM_SHARED((NPAD, D), jnp.float32),
        ]
        + [pltpu.SemaphoreType.DMA] * (4 * NB),
    )
    def k(h_hbm, src_hbm, dst_hbm, out_hbm, srcv, dstv, rows_v, zbuf, acc_sh,
          *sems):
        gsem = sems[:NB]
        ssem = sems[NB:2 * NB]
        isem_s = sems[2 * NB:3 * NB]
        isem_d = sems[3 * NB:]
        cid = lax.axis_index("c")
        sid = lax.axis_index("s")
        wid = sid * NC + cid

        def is_start(b, ci):
            pltpu.async_copy(src_hbm.at[wid].at[ci], srcv.at[b], isem_s[b])

        def is_wait(b, ci):
            pltpu.make_async_copy(src_hbm.at[wid].at[ci], srcv.at[b],
                                  isem_s[b]).wait()

        def id_start(b, ci):
            pltpu.async_copy(dst_hbm.at[wid].at[ci], dstv.at[b], isem_d[b])

        def id_wait(b, ci):
            pltpu.make_async_copy(dst_hbm.at[wid].at[ci], dstv.at[b],
                                  isem_d[b]).wait()

        def g_start(b):
            pltpu.async_copy(h_hbm.at[srcv.at[b]], rows_v.at[b], gsem[b])

        def g_wait(b):
            pltpu.make_async_copy(h_hbm.at[srcv.at[b]], rows_v.at[b],
                                  gsem[b]).wait()

        def s_start(b):
            pltpu.async_copy(rows_v.at[b], acc_sh.at[dstv.at[b]], ssem[b],
                             add=True)

        def s_wait(b):
            pltpu.make_async_copy(rows_v.at[b], acc_sh.at[dstv.at[b]],
                                  ssem[b]).wait()

        for b in range(NB):
            is_start(b, b)
            id_start(b, b)
        for b in range(NB):
            is_wait(b, b)
            g_start(b)

        zbuf[...] = jnp.zeros((ZROWS, D), jnp.float32)

        @pl.loop(0, RPS, step=ZROWS)
        def _(r):
            pltpu.sync_copy(zbuf, acc_sh.at[pl.ds(sid * RPS + r, ZROWS)])

        plsc.subcore_barrier()

        @pl.loop(0, NCHUNK - NB, step=NB)
        def _(c0):
            for b in range(NB):
                g_wait(b)
                is_start(b, c0 + NB + b)
                id_wait(b, c0 + b)
                s_start(b)
            for b in range(NB):
                s_wait(b)
                id_start(b, c0 + NB + b)
                is_wait(b, c0 + NB + b)
                g_start(b)

        for b in range(NB):
            g_wait(b)
            id_wait(b, NCHUNK - NB + b)
            s_start(b)
        for b in range(NB):
            s_wait(b)

        plsc.subcore_barrier()

        @pl.when(sid < NS - 1)
        def _():
            pltpu.sync_copy(acc_sh.at[pl.ds(sid * RPS, RPS)],
                            out_hbm.at[cid].at[pl.ds(sid * RPS, RPS)])

        @pl.when(sid == NS - 1)
        def _():
            pltpu.sync_copy(acc_sh.at[pl.ds((NS - 1) * RPS, N - (NS - 1) * RPS)],
                            out_hbm.at[cid].at[pl.ds((NS - 1) * RPS, N - (NS - 1) * RPS)])

    return k(h, src3, dst3)


def _dot(a, b_t):
    return lax.dot_general(a, b_t, (((1,), (1,)), ((), ())),
                           precision=lax.Precision.HIGHEST,
                           preferred_element_type=jnp.float32)


def _tc_layer(agg, h, w_rel, b_rel, w_root, relu):
    def body(agg_ref, h_ref, wr_ref, b_ref, wo_ref, o_ref):
        a = agg_ref[0] + agg_ref[1]
        o = _dot(a, wr_ref[...]) + _dot(h_ref[...], wo_ref[...]) + b_ref[...]
        if relu:
            o = jnp.maximum(o, 0.0)
        o_ref[...] = o

    return pl.pallas_call(
        body,
        grid=(GRID,),
        in_specs=[
            pl.BlockSpec((2, BN, D), lambda i: (0, i, 0)),
            pl.BlockSpec((BN, D), lambda i: (i, 0)),
            pl.BlockSpec((H, D), lambda i: (0, 0)),
            pl.BlockSpec((1, H), lambda i: (0, 0)),
            pl.BlockSpec((H, D), lambda i: (0, 0)),
        ],
        out_specs=pl.BlockSpec((BN, H), lambda i: (i, 0)),
        out_shape=jax.ShapeDtypeStruct((N, H), jnp.float32),
    )(agg, h, w_rel, b_rel.reshape(1, H), w_root)


def _tc_final(agg, h, w_rel, b_rel, w_root, batch, lin_w, lin_b):
    def body(agg_ref, h_ref, wr_ref, b_ref, wo_ref, batch_ref, lw_ref, lb_ref,
             o_ref, sums, cnts):
        i = pl.program_id(0)
        a = agg_ref[0] + agg_ref[1]
        h3 = _dot(a, wr_ref[...]) + _dot(h_ref[...], wo_ref[...]) + b_ref[...]
        seg = batch_ref[...]
        onehot = (seg == lax.broadcasted_iota(jnp.int32, (BN, G), 1)
                  ).astype(jnp.float32)
        part = lax.dot_general(onehot, h3, (((0,), (0,)), ((), ())),
                               precision=lax.Precision.HIGHEST,
                               preferred_element_type=jnp.float32)
        pcnt = lax.dot_general(onehot, jnp.ones((BN, D), jnp.float32),
                               (((0,), (0,)), ((), ())),
                               precision=lax.Precision.HIGHEST,
                               preferred_element_type=jnp.float32)

        @pl.when(i == 0)
        def _():
            sums[...] = part
            cnts[...] = pcnt

        @pl.when(i > 0)
        def _():
            sums[...] += part
            cnts[...] += pcnt

        @pl.when(i == GRID - 1)
        def _():
            pooled = sums[...] / jnp.maximum(cnts[...], 1.0)
            o_ref[...] = _dot(pooled, lw_ref[...]) + lb_ref[...]

    return pl.pallas_call(
        body,
        grid=(GRID,),
        in_specs=[
            pl.BlockSpec((2, BN, D), lambda i: (0, i, 0)),
            pl.BlockSpec((BN, D), lambda i: (i, 0)),
            pl.BlockSpec((H, D), lambda i: (0, 0)),
            pl.BlockSpec((1, H), lambda i: (0, 0)),
            pl.BlockSpec((H, D), lambda i: (0, 0)),
            pl.BlockSpec((BN, 1), lambda i: (i, 0)),
            pl.BlockSpec((C, H), lambda i: (0, 0)),
            pl.BlockSpec((1, C), lambda i: (0, 0)),
        ],
        out_specs=pl.BlockSpec((G, C), lambda i: (0, 0)),
        out_shape=jax.ShapeDtypeStruct((G, C), jnp.float32),
        scratch_shapes=[
            pltpu.VMEM((G, D), jnp.float32),
            pltpu.VMEM((G, D), jnp.float32),
        ],
    )(agg, h, w_rel, b_rel.reshape(1, H), w_root, batch.reshape(N, 1),
      lin_w, lin_b.reshape(1, C))


def kernel(x, edge_index, batch, W1_rel, b1_rel, W1_root, W2_rel, b2_rel,
           W2_root, W3_rel, b3_rel, W3_root, lin_W, lin_b):
    src = edge_index[0]
    dst = edge_index[1]
    pad = EPAD - E
    trash = N + jax.lax.iota(jnp.int32, pad) % (NPAD - N)
    src3 = jnp.concatenate([src, jnp.zeros((pad,), jnp.int32)]
                           ).reshape(NW, NCHUNK, CHUNK)
    dst3 = jnp.concatenate([dst, trash]).reshape(NW, NCHUNK, CHUNK)

    agg1 = _sc_segsum(x, src3, dst3)
    h1 = _tc_layer(agg1, x, W1_rel, b1_rel, W1_root, relu=True)
    agg2 = _sc_segsum(h1, src3, dst3)
    h2 = _tc_layer(agg2, h1, W2_rel, b2_rel, W2_root, relu=True)
    agg3 = _sc_segsum(h2, src3, dst3)
    return _tc_final(agg3, h2, W3_rel, b3_rel, W3_root, batch, lin_W, lin_b)
